# baseline (speedup 1.0000x reference)
import functools

import jax
import jax.numpy as jnp
from jax import lax
from jax.experimental import pallas as pl
from jax.experimental.pallas import tpu as pltpu
from jax.experimental.pallas import tpu_sc as plsc

N_ROWS = 1000000
EMBED_DIM = 64
BATCH = 16384
NUM_CORES = 2
NUM_SUBCORES = 16
NUM_WORKERS = NUM_CORES * NUM_SUBCORES
LANE = 16
GROUP = 512
N_GROUPS = 1954
WAVES = 62
CAP = 640
N_CH = CAP // LANE
SENTINEL_J = 1 << 20


@functools.partial(
    pl.kernel,
    out_type=jax.ShapeDtypeStruct((BATCH + 8, EMBED_DIM), jnp.float32),
    mesh=plsc.VectorSubcoreMesh(core_axis_name="c", subcore_axis_name="s"),
    compiler_params=pltpu.CompilerParams(needs_layout_passes=False),
    scratch_types=[
        pltpu.VMEM((BATCH // 2,), jnp.int32),
        pltpu.VMEM((CAP,), jnp.int32),
        pltpu.VMEM((CAP,), jnp.int32),
        pltpu.VMEM((EMBED_DIM, GROUP), jnp.float32),
        pltpu.VMEM((CAP, EMBED_DIM), jnp.float32),
        pltpu.SemaphoreType.DMA,
        pltpu.SemaphoreType.DMA,
    ],
)
def _sc_gather(idx_hbm, table_t_hbm, out_hbm, idx_v, mj_v, mp_v, buf_v,
               rows_v, sem_i, sem):
    wid = lax.axis_index("s") * NUM_CORES + lax.axis_index("c")

    for ch in range(N_CH):
        mj_v[pl.ds(ch * LANE, LANE)] = jnp.full((LANE,), SENTINEL_J, jnp.int32)
        mp_v[pl.ds(ch * LANE, LANE)] = jnp.full((LANE,), BATCH, jnp.int32)

    off = jnp.int32(0)
    for h in range(2):
        pltpu.async_copy(
            idx_hbm.at[pl.ds(h * (BATCH // 2), BATCH // 2)], idx_v, sem_i
        ).wait()

        def compact(i, off, h=h):
            jv = idx_v[pl.ds(i * LANE, LANE)]
            grp = jv >> 9
            m = (grp & 31) == wid
            cs = plsc.cumsum(jnp.where(m, 1, 0))
            dst = jnp.minimum(off + cs - 1, CAP - 1)
            plsc.store_scatter(mj_v, [dst], jv, mask=m)
            pos = lax.iota(jnp.int32, LANE) + (h * (BATCH // 2) + i * LANE)
            plsc.store_scatter(mp_v, [dst], pos, mask=m)
            return off + cs[LANE - 1]
        off = lax.fori_loop(0, BATCH // 2 // LANE, compact, off)

    def wave(step):
        g = step * NUM_WORKERS + wid
        lanes = g * GROUP

        @pl.when(g < N_GROUPS - 1)
        def _full():
            col = pl.multiple_of(lanes, 128)
            pltpu.async_copy(
                table_t_hbm.at[:, pl.ds(col, GROUP)], buf_v, sem
            ).wait()

        @pl.when(g == N_GROUPS - 1)
        def _tail():
            col = pl.multiple_of(lanes, 128)
            pltpu.async_copy(
                table_t_hbm.at[:, pl.ds(col, 128)],
                buf_v.at[:, pl.ds(0, 128)],
                sem,
            ).wait()

        def scan(ch):
            jv = mj_v[pl.ds(ch * LANE, LANE)]
            m = (jv >> 9) == g
            mi = jnp.where(m, 1, 0)
            cnt = plsc.all_reduce_population_count(m)

            @pl.when(cnt[0] > 0)
            def _hits():
                for b in range(LANE):
                    @pl.when(mi[b] == 1)
                    def _one(b=b):
                        jloc = jnp.full((LANE,), jv[b] & 511, jnp.int32)
                        row = ch * LANE + b
                        for k in range(EMBED_DIM // LANE):
                            ev = lax.iota(jnp.int32, LANE) + (k * LANE)
                            rows_v[row, pl.ds(k * LANE, LANE)] = (
                                plsc.load_gather(buf_v, [ev, jloc])
                            )
        pl.loop(0, N_CH)(scan)
    pl.loop(0, WAVES)(wave)

    def flush(ch):
        pv = mp_v[pl.ds(ch * LANE, LANE)]
        cps = []
        for b in range(LANE):
            cps.append(
                pltpu.async_copy(
                    rows_v.at[ch * LANE + b], out_hbm.at[pv[b]], sem
                )
            )
        for cp in cps:
            cp.wait()
    pl.loop(0, N_CH)(flush)


def kernel(action_idx, embedding_weight):
    idx = action_idx.astype(jnp.int32)
    out = _sc_gather(idx, embedding_weight.T)
    return out[:BATCH]


# speedup vs baseline: 1.7768x; 1.7768x over previous
import functools

import jax
import jax.numpy as jnp
from jax import lax
from jax.experimental import pallas as pl
from jax.experimental.pallas import tpu as pltpu
from jax.experimental.pallas import tpu_sc as plsc

N_ROWS = 1000000
EMBED_DIM = 64
BATCH = 16384
NUM_CORES = 2
NUM_SUBCORES = 16
NUM_WORKERS = NUM_CORES * NUM_SUBCORES
LANE = 16
GROUP = 512
N_GROUPS = 1954
WAVES = 62
CAP = 640
N_CH = CAP // LANE
SENTINEL_J = 1 << 20


@functools.partial(
    pl.kernel,
    out_type=jax.ShapeDtypeStruct((BATCH + 8, EMBED_DIM), jnp.float32),
    mesh=plsc.VectorSubcoreMesh(core_axis_name="c", subcore_axis_name="s"),
    compiler_params=pltpu.CompilerParams(needs_layout_passes=False),
    scratch_types=[
        pltpu.VMEM((BATCH // 2,), jnp.int32),
        pltpu.VMEM((CAP,), jnp.int32),
        pltpu.VMEM((CAP,), jnp.int32),
        pltpu.VMEM((EMBED_DIM, GROUP), jnp.float32),
        pltpu.VMEM((CAP, EMBED_DIM), jnp.float32),
        pltpu.SemaphoreType.DMA,
        pltpu.SemaphoreType.DMA,
    ],
)
def _sc_gather(idx_hbm, table_t_hbm, out_hbm, idx_v, mj_v, mp_v, buf_v,
               rows_v, sem_i, sem):
    wid = lax.axis_index("s") * NUM_CORES + lax.axis_index("c")

    for ch in range(N_CH):
        mj_v[pl.ds(ch * LANE, LANE)] = jnp.full((LANE,), SENTINEL_J, jnp.int32)
        mp_v[pl.ds(ch * LANE, LANE)] = jnp.full((LANE,), BATCH, jnp.int32)

    off = jnp.int32(0)
    for h in range(2):
        pltpu.async_copy(
            idx_hbm.at[pl.ds(h * (BATCH // 2), BATCH // 2)], idx_v, sem_i
        ).wait()

        def compact(i, off, h=h):
            jv = idx_v[pl.ds(i * LANE, LANE)]
            grp = jv >> 9
            m = (grp & 31) == wid
            cs = plsc.cumsum(jnp.where(m, 1, 0))
            dst = jnp.minimum(off + cs - 1, CAP - 1)
            plsc.store_scatter(mj_v, [dst], jv, mask=m)
            pos = lax.iota(jnp.int32, LANE) + (h * (BATCH // 2) + i * LANE)
            plsc.store_scatter(mp_v, [dst], pos, mask=m)
            return off + cs[LANE - 1]
        off = lax.fori_loop(0, BATCH // 2 // LANE, compact, off)

    def wave(step):
        g = step * NUM_WORKERS + wid
        lanes = g * GROUP

        @pl.when(g < N_GROUPS - 1)
        def _full():
            col = pl.multiple_of(lanes, 128)
            pltpu.async_copy(
                table_t_hbm.at[:, pl.ds(col, GROUP)], buf_v, sem
            ).wait()

        @pl.when(g == N_GROUPS - 1)
        def _tail():
            col = pl.multiple_of(lanes, 128)
            pltpu.async_copy(
                table_t_hbm.at[:, pl.ds(col, 128)],
                buf_v.at[:, pl.ds(0, 128)],
                sem,
            ).wait()

        def scan(ch):
            jv = mj_v[pl.ds(ch * LANE, LANE)]
            m = (jv >> 9) == g
            mi = jnp.where(m, 1, 0)
            cnt = plsc.all_reduce_population_count(m)

            @pl.when(cnt[0] > 0)
            def _hits():
                for b in range(LANE):
                    @pl.when(mi[b] == 1)
                    def _one(b=b):
                        jloc = jnp.full((LANE,), jv[b] & 511, jnp.int32)
                        row = ch * LANE + b
                        for k in range(EMBED_DIM // LANE):
                            ev = lax.iota(jnp.int32, LANE) + (k * LANE)
                            rows_v[row, pl.ds(k * LANE, LANE)] = (
                                plsc.load_gather(buf_v, [ev, jloc])
                            )
        pl.loop(0, 1)(scan)
    pl.loop(0, WAVES)(wave)

    def flush(ch):
        pv = mp_v[pl.ds(ch * LANE, LANE)]
        cps = []
        for b in range(LANE):
            cps.append(
                pltpu.async_copy(
                    rows_v.at[ch * LANE + b], out_hbm.at[pv[b]], sem
                )
            )
        for cp in cps:
            cp.wait()
    pl.loop(0, N_CH)(flush)


def kernel(action_idx, embedding_weight):
    idx = action_idx.astype(jnp.int32)
    out = _sc_gather(idx, embedding_weight.T)
    return out[:BATCH]


# speedup vs baseline: 1.9877x; 1.1187x over previous
import functools

import jax
import jax.numpy as jnp
from jax import lax
from jax.experimental import pallas as pl
from jax.experimental.pallas import tpu as pltpu
from jax.experimental.pallas import tpu_sc as plsc

N_ROWS = 1000000
EMBED_DIM = 64
BATCH = 16384
NUM_CORES = 2
NUM_SUBCORES = 16
NUM_WORKERS = NUM_CORES * NUM_SUBCORES
B_PER_W = BATCH // NUM_WORKERS
LANE = 16
CHUNK = 128
N_CHUNKS = B_PER_W // CHUNK
FIRE = 16
RING = 4


@functools.partial(
    pl.kernel,
    out_type=jax.ShapeDtypeStruct((BATCH, EMBED_DIM), jnp.float32),
    mesh=plsc.VectorSubcoreMesh(core_axis_name="c", subcore_axis_name="s"),
    compiler_params=pltpu.CompilerParams(needs_layout_passes=False),
    scratch_types=[
        pltpu.VMEM((N_CHUNKS, CHUNK), jnp.int32),
        pltpu.VMEM((EMBED_DIM, CHUNK), jnp.float32),
        pltpu.VMEM((EMBED_DIM, CHUNK), jnp.float32),
        pltpu.VMEM((EMBED_DIM, CHUNK), jnp.float32),
        pltpu.VMEM((EMBED_DIM, CHUNK), jnp.float32),
        pltpu.VMEM((B_PER_W, EMBED_DIM), jnp.float32),
        pltpu.SemaphoreType.DMA,
        pltpu.SemaphoreType.DMA,
    ],
)
def _sc_gather(idx_hbm, table_t_hbm, out_hbm, idx_v, col0, col1, col2, col3,
               rows_v, sem_i, sem):
    cols = (col0, col1, col2, col3)
    wid = lax.axis_index("s") * NUM_CORES + lax.axis_index("c")
    base = wid * B_PER_W

    pltpu.async_copy(idx_hbm.at[wid], idx_v, sem_i).wait()

    def body(step, c):
        p = step * FIRE
        jv = idx_v[c, pl.ds(p, FIRE)]
        for q in range(FIRE // RING):
            cps = []
            for b in range(RING):
                j = jv[q * RING + b]
                col = pl.multiple_of((j >> 7) * CHUNK, CHUNK)
                cps.append(
                    pltpu.async_copy(
                        table_t_hbm.at[:, pl.ds(col, CHUNK)], cols[b], sem
                    )
                )
            for b in range(RING):
                cps[b].wait()
                j = jv[q * RING + b]
                jl = jnp.full((LANE,), j & 127, jnp.int32)
                row = c * CHUNK + p + q * RING + b
                for k in range(EMBED_DIM // LANE):
                    ev = lax.iota(jnp.int32, LANE) + (k * LANE)
                    rows_v[row, pl.ds(k * LANE, LANE)] = plsc.load_gather(
                        cols[b], [ev, jl]
                    )
    for c in range(N_CHUNKS):
        pl.loop(0, CHUNK // FIRE)(functools.partial(body, c=c))

    pltpu.async_copy(rows_v, out_hbm.at[pl.ds(base, B_PER_W)], sem_i).wait()


def kernel(action_idx, embedding_weight):
    idx = action_idx.astype(jnp.int32).reshape(NUM_WORKERS, N_CHUNKS, CHUNK)
    return _sc_gather(idx, embedding_weight.T)


# speedup vs baseline: 2.3907x; 1.2028x over previous
import functools

import jax
import jax.numpy as jnp
from jax import lax
from jax.experimental import pallas as pl
from jax.experimental.pallas import tpu as pltpu
from jax.experimental.pallas import tpu_sc as plsc

N_ROWS = 1000000
EMBED_DIM = 64
BATCH = 16384
NUM_CORES = 2
NUM_SUBCORES = 16
NUM_WORKERS = NUM_CORES * NUM_SUBCORES
B_PER_W = BATCH // NUM_WORKERS
LANE = 16
CHUNK = 128
N_CHUNKS = B_PER_W // CHUNK
FIRE = 16
RING = 8


@functools.partial(
    pl.kernel,
    out_type=jax.ShapeDtypeStruct((BATCH, EMBED_DIM), jnp.float32),
    mesh=plsc.VectorSubcoreMesh(core_axis_name="c", subcore_axis_name="s"),
    compiler_params=pltpu.CompilerParams(needs_layout_passes=False),
    scratch_types=[
        pltpu.VMEM((N_CHUNKS, CHUNK), jnp.int32),
        pltpu.VMEM((EMBED_DIM, CHUNK), jnp.float32),
        pltpu.VMEM((EMBED_DIM, CHUNK), jnp.float32),
        pltpu.VMEM((EMBED_DIM, CHUNK), jnp.float32),
        pltpu.VMEM((EMBED_DIM, CHUNK), jnp.float32),
        pltpu.VMEM((EMBED_DIM, CHUNK), jnp.float32),
        pltpu.VMEM((EMBED_DIM, CHUNK), jnp.float32),
        pltpu.VMEM((EMBED_DIM, CHUNK), jnp.float32),
        pltpu.VMEM((EMBED_DIM, CHUNK), jnp.float32),
        pltpu.VMEM((CHUNK, EMBED_DIM), jnp.float32),
        pltpu.SemaphoreType.DMA,
        pltpu.SemaphoreType.DMA,
    ],
)
def _sc_gather(idx_hbm, table_t_hbm, out_hbm, idx_v, col0, col1, col2, col3,
               col4, col5, col6, col7, rows_v, sem_i, sem):
    cols = (col0, col1, col2, col3, col4, col5, col6, col7)
    wid = lax.axis_index("s") * NUM_CORES + lax.axis_index("c")
    base = wid * B_PER_W

    pltpu.async_copy(idx_hbm.at[wid], idx_v, sem_i).wait()

    def body(step, c):
        p = step * FIRE
        jv = idx_v[c, pl.ds(p, FIRE)]
        for q in range(FIRE // RING):
            cps = []
            for b in range(RING):
                j = jv[q * RING + b]
                col = pl.multiple_of((j >> 7) * CHUNK, CHUNK)
                cps.append(
                    pltpu.async_copy(
                        table_t_hbm.at[:, pl.ds(col, CHUNK)], cols[b], sem
                    )
                )
            for b in range(RING):
                cps[b].wait()
                j = jv[q * RING + b]
                jl = jnp.full((LANE,), j & 127, jnp.int32)
                row = p + q * RING + b
                for k in range(EMBED_DIM // LANE):
                    ev = lax.iota(jnp.int32, LANE) + (k * LANE)
                    rows_v[row, pl.ds(k * LANE, LANE)] = plsc.load_gather(
                        cols[b], [ev, jl]
                    )
    for c in range(N_CHUNKS):
        pl.loop(0, CHUNK // FIRE)(functools.partial(body, c=c))
        pltpu.async_copy(
            rows_v, out_hbm.at[pl.ds(base + c * CHUNK, CHUNK)], sem_i
        ).wait()


def kernel(action_idx, embedding_weight):
    idx = action_idx.astype(jnp.int32).reshape(NUM_WORKERS, N_CHUNKS, CHUNK)
    return _sc_gather(idx, embedding_weight.T)


# speedup vs baseline: 2.6240x; 1.0976x over previous
import functools

import jax
import jax.numpy as jnp
from jax import lax
from jax.experimental import pallas as pl
from jax.experimental.pallas import tpu as pltpu
from jax.experimental.pallas import tpu_sc as plsc

N_ROWS = 1000000
EMBED_DIM = 64
BATCH = 16384
NUM_CORES = 2
NUM_SUBCORES = 16
NUM_WORKERS = NUM_CORES * NUM_SUBCORES
B_PER_W = BATCH // NUM_WORKERS
LANE = 16
CHUNK = 128
N_CHUNKS = B_PER_W // CHUNK
FIRE = 16
RING = 10


@functools.partial(
    pl.kernel,
    out_type=jax.ShapeDtypeStruct((BATCH, EMBED_DIM), jnp.float32),
    mesh=plsc.VectorSubcoreMesh(core_axis_name="c", subcore_axis_name="s"),
    compiler_params=pltpu.CompilerParams(needs_layout_passes=False),
    scratch_types=[
        pltpu.VMEM((N_CHUNKS, CHUNK), jnp.int32),
        pltpu.VMEM((EMBED_DIM, CHUNK), jnp.float32),
        pltpu.VMEM((EMBED_DIM, CHUNK), jnp.float32),
        pltpu.VMEM((EMBED_DIM, CHUNK), jnp.float32),
        pltpu.VMEM((EMBED_DIM, CHUNK), jnp.float32),
        pltpu.VMEM((EMBED_DIM, CHUNK), jnp.float32),
        pltpu.VMEM((EMBED_DIM, CHUNK), jnp.float32),
        pltpu.VMEM((EMBED_DIM, CHUNK), jnp.float32),
        pltpu.VMEM((EMBED_DIM, CHUNK), jnp.float32),
        pltpu.VMEM((EMBED_DIM, CHUNK), jnp.float32),
        pltpu.VMEM((EMBED_DIM, CHUNK), jnp.float32),
        pltpu.VMEM((CHUNK, EMBED_DIM), jnp.float32),
        pltpu.SemaphoreType.DMA,
        pltpu.SemaphoreType.DMA,
    ],
)
def _sc_gather(idx_hbm, table_t_hbm, out_hbm, idx_v, col0, col1, col2, col3,
               col4, col5, col6, col7, col8, col9, rows_v, sem_i, sem):
    cols = (col0, col1, col2, col3, col4, col5, col6, col7, col8, col9)
    wid = lax.axis_index("s") * NUM_CORES + lax.axis_index("c")
    base = wid * B_PER_W

    pltpu.async_copy(idx_hbm.at[wid], idx_v, sem_i).wait()

    def fire(jv, i, sem):
        j = jv[i]
        col = pl.multiple_of((j >> 7) * CHUNK, CHUNK)
        return pltpu.async_copy(
            table_t_hbm.at[:, pl.ds(col, CHUNK)], cols[i % RING], sem
        )

    def body(step, c):
        p = step * FIRE
        jv = idx_v[c, pl.ds(p, FIRE)]
        cps = [fire(jv, i, sem) for i in range(RING)]
        for i in range(FIRE):
            cps[i].wait()
            j = jv[i]
            jl = jnp.full((LANE,), j & 127, jnp.int32)
            for k in range(EMBED_DIM // LANE):
                ev = lax.iota(jnp.int32, LANE) + (k * LANE)
                rows_v[p + i, pl.ds(k * LANE, LANE)] = plsc.load_gather(
                    cols[i % RING], [ev, jl]
                )
            if i + RING < FIRE:
                cps.append(fire(jv, i + RING, sem))
    for c in range(N_CHUNKS):
        pl.loop(0, CHUNK // FIRE)(functools.partial(body, c=c))
        pltpu.async_copy(
            rows_v, out_hbm.at[pl.ds(base + c * CHUNK, CHUNK)], sem_i
        ).wait()


def kernel(action_idx, embedding_weight):
    idx = action_idx.astype(jnp.int32).reshape(NUM_WORKERS, N_CHUNKS, CHUNK)
    return _sc_gather(idx, embedding_weight.T)
